# baseline (speedup 1.0000x reference)
import functools

import jax
import jax.numpy as jnp
from jax import lax
from jax.experimental import pallas as pl
from jax.experimental.pallas import tpu as pltpu
from jax.experimental.pallas import tpu_sc as plsc

NC = 2
NS = 16
L = 16
NWRIT = 16


def _make_deg_kernel(E, NP):
    EW = E // (NC * NS)
    NB = 5
    RED = NP // NS
    mesh = plsc.VectorSubcoreMesh(
        core_axis_name="c", subcore_axis_name="s", num_cores=NC, num_subcores=NS
    )

    @functools.partial(
        pl.kernel,
        out_type=(
            jax.ShapeDtypeStruct((NP,), jnp.float32),
            jax.ShapeDtypeStruct((NP,), jnp.float32),
        ),
        mesh=mesh,
        compiler_params=pltpu.CompilerParams(needs_layout_passes=False),
        scratch_types=[
            [pltpu.VMEM((EW // NB,), jnp.int32)] * 2,
            [pltpu.VMEM((EW // NB,), jnp.float32)] * 2,
            pltpu.VMEM((NP,), jnp.float32),
            pltpu.VMEM((RED,), jnp.float32),
            pltpu.VMEM((RED,), jnp.float32),
            pltpu.VMEM((RED,), jnp.float32),
            pltpu.VMEM_SHARED((NS, NP), jnp.float32),
            [pltpu.SemaphoreType.DMA] * 2,
            [pltpu.SemaphoreType.DMA] * 2,
        ],
    )
    def deg_kernel(ei_hbm, w_hbm, out0_hbm, out1_hbm, row_b, w_b, deg_v,
                   red_v, tmp0_v, tmp1_v, shared, sem_r, sem_w):
        c = lax.axis_index("c")
        s = lax.axis_index("s")
        wid = s * NC + c
        BL = EW // NB

        def fetch(blk):
            p = blk % 2
            base = wid * EW + blk * BL
            return (
                pltpu.async_copy(ei_hbm.at[pl.ds(base, BL)], row_b[p], sem_r[p]),
                pltpu.async_copy(w_hbm.at[pl.ds(base, BL)], w_b[p], sem_w[p]),
            )

        cps = fetch(0)

        @plsc.parallel_loop(0, NP // L, 1, unroll=8)
        def _(i):
            deg_v[pl.ds(i * L, L)] = jnp.zeros((L,), jnp.float32)

        for blk in range(NB):
            for cp in cps:
                cp.wait()
            if blk + 1 < NB:
                cps = fetch(blk + 1)
            p = blk % 2
            row_v, w_v = row_b[p], w_b[p]

            @plsc.parallel_loop(0, BL // L, 1, unroll=5)
            def _(i, row_v=row_v, w_v=w_v):
                o = pl.ds(i * L, L)
                plsc.addupdate_scatter(deg_v, [row_v[o]], w_v[o])

        pltpu.sync_copy(deg_v, shared.at[s])
        plsc.subcore_barrier()

        slr = pl.ds(s * RED, RED)
        pltpu.sync_copy(shared.at[0, slr], red_v)

        def red_body(j, _):
            t = 1 + j * 2
            pltpu.async_copy(shared.at[t, slr], tmp0_v, sem_r[0])
            pltpu.async_copy(shared.at[t + 1, slr], tmp1_v, sem_r[1])
            pltpu.make_async_copy(shared.at[t, slr], tmp0_v, sem_r[0]).wait()

            @plsc.parallel_loop(0, RED // L, 1, unroll=4)
            def _(k):
                plsc.addupdate(red_v.at[pl.ds(k * L, L)], tmp0_v[pl.ds(k * L, L)])
            pltpu.make_async_copy(shared.at[t + 1, slr], tmp1_v, sem_r[1]).wait()

            @plsc.parallel_loop(0, RED // L, 1, unroll=4)
            def _(k):
                plsc.addupdate(red_v.at[pl.ds(k * L, L)], tmp1_v[pl.ds(k * L, L)])
            return ()
        lax.fori_loop(0, (NS - 1) // 2, red_body, ())
        pltpu.sync_copy(shared.at[NS - 1, slr], tmp0_v)

        @plsc.parallel_loop(0, RED // L, 1, unroll=4)
        def _(k):
            plsc.addupdate(red_v.at[pl.ds(k * L, L)], tmp0_v[pl.ds(k * L, L)])

        @pl.when(c == 0)
        def _():
            pltpu.sync_copy(red_v, out0_hbm.at[slr])

        @pl.when(c == 1)
        def _():
            pltpu.sync_copy(red_v, out1_hbm.at[slr])

    return deg_kernel


def _make_agg_kernel(E, NP2):
    EW = E // (NC * NS)
    NB = 5
    RED = NP2 // NS
    mesh = plsc.VectorSubcoreMesh(
        core_axis_name="c", subcore_axis_name="s", num_cores=NC, num_subcores=NS
    )

    @functools.partial(
        pl.kernel,
        out_type=(
            jax.ShapeDtypeStruct((NP2,), jnp.float32),
            jax.ShapeDtypeStruct((NP2,), jnp.float32),
        ),
        mesh=mesh,
        compiler_params=pltpu.CompilerParams(needs_layout_passes=False),
        scratch_types=[
            [pltpu.VMEM((EW // NB,), jnp.int32)] * 2,
            [pltpu.VMEM((EW // NB,), jnp.int32)] * 2,
            [pltpu.VMEM((EW // NB,), jnp.float32)] * 2,
            pltpu.VMEM((NP2,), jnp.float32),
            pltpu.VMEM((NP2,), jnp.float32),
            pltpu.VMEM((RED,), jnp.float32),
            pltpu.VMEM((RED,), jnp.float32),
            pltpu.VMEM((RED,), jnp.float32),
            pltpu.VMEM_SHARED((NS, NP2), jnp.float32),
            [pltpu.SemaphoreType.DMA] * 2,
            [pltpu.SemaphoreType.DMA] * 2,
            [pltpu.SemaphoreType.DMA] * 2,
            pltpu.SemaphoreType.DMA,
        ],
    )
    def agg_kernel(ei_hbm, w_hbm, g2_hbm, out0_hbm, out1_hbm,
                   row_b, col_b, w_b, g2_v, acc_v, red_v, tmp0_v, tmp1_v,
                   shared, sem_r, sem_cc, sem_w, sem_g):
        c = lax.axis_index("c")
        s = lax.axis_index("s")
        wid = s * NC + c
        BL = EW // NB

        cp_g = pltpu.async_copy(g2_hbm, g2_v, sem_g)

        def fetch(blk):
            p = blk % 2
            base = wid * EW + blk * BL
            return (
                pltpu.async_copy(ei_hbm.at[pl.ds(base, BL)], row_b[p], sem_r[p]),
                pltpu.async_copy(ei_hbm.at[pl.ds(E + base, BL)], col_b[p],
                                 sem_cc[p]),
                pltpu.async_copy(w_hbm.at[pl.ds(base, BL)], w_b[p], sem_w[p]),
            )

        cps = fetch(0)

        @plsc.parallel_loop(0, NP2 // L, 1, unroll=8)
        def _(i):
            acc_v[pl.ds(i * L, L)] = jnp.zeros((L,), jnp.float32)
        cp_g.wait()

        npad = jnp.int32(NP2 // 2)
        for blk in range(NB):
            for cp in cps:
                cp.wait()
            if blk + 1 < NB:
                cps = fetch(blk + 1)
            p = blk % 2
            row_v, col_v, w_v = row_b[p], col_b[p], w_b[p]

            @plsc.parallel_loop(0, BL // L, 1, unroll=5)
            def _(i, row_v=row_v, col_v=col_v, w_v=w_v):
                o = pl.ds(i * L, L)
                r = row_v[o]
                cc2 = col_v[o]
                wt = w_v[o]
                g0 = plsc.load_gather(g2_v, [r])
                g1 = plsc.load_gather(g2_v, [r + npad])
                plsc.addupdate_scatter(acc_v, [cc2], g0 * wt)
                plsc.addupdate_scatter(acc_v, [cc2 + npad], g1 * wt)

        pltpu.sync_copy(acc_v, shared.at[s])
        plsc.subcore_barrier()

        slr = pl.ds(s * RED, RED)
        pltpu.sync_copy(shared.at[0, slr], red_v)

        def red_body(j, _):
            t = 1 + j * 2
            pltpu.async_copy(shared.at[t, slr], tmp0_v, sem_r[0])
            pltpu.async_copy(shared.at[t + 1, slr], tmp1_v, sem_r[1])
            pltpu.make_async_copy(shared.at[t, slr], tmp0_v, sem_r[0]).wait()

            @plsc.parallel_loop(0, RED // L, 1, unroll=4)
            def _(k):
                plsc.addupdate(red_v.at[pl.ds(k * L, L)], tmp0_v[pl.ds(k * L, L)])
            pltpu.make_async_copy(shared.at[t + 1, slr], tmp1_v, sem_r[1]).wait()

            @plsc.parallel_loop(0, RED // L, 1, unroll=4)
            def _(k):
                plsc.addupdate(red_v.at[pl.ds(k * L, L)], tmp1_v[pl.ds(k * L, L)])
            return ()
        lax.fori_loop(0, (NS - 1) // 2, red_body, ())
        pltpu.sync_copy(shared.at[NS - 1, slr], tmp0_v)

        @plsc.parallel_loop(0, RED // L, 1, unroll=4)
        def _(k):
            plsc.addupdate(red_v.at[pl.ds(k * L, L)], tmp0_v[pl.ds(k * L, L)])

        @pl.when(c == 0)
        def _():
            pltpu.sync_copy(red_v, out0_hbm.at[slr])

        @pl.when(c == 1)
        def _():
            pltpu.sync_copy(red_v, out1_hbm.at[slr])

    return agg_kernel


def _tc_norm_body(d0_ref, d1_ref, x_ref, w_ref, g_ref, dv_ref):
    NPAD = d0_ref.shape[0]
    N = x_ref.shape[0]
    deg = d0_ref[...] + d1_ref[...] + 1.0
    dinv = lax.rsqrt(jnp.maximum(deg, 1e-12))
    dv_ref[...] = dinv
    hT = lax.dot_general(
        w_ref[...], x_ref[...],
        dimension_numbers=(((0,), (1,)), ((), ())),
        preferred_element_type=jnp.float32,
    )
    gT = hT * dinv[None, :N]
    C = gT.shape[0]
    for cc in range(C):
        g_ref[pl.ds(cc * NPAD, N)] = gT[cc]
        g_ref[pl.ds(cc * NPAD + N, NPAD - N)] = jnp.zeros(
            (NPAD - N,), jnp.float32)


def _tc_out_body(a0_ref, a1_ref, g_ref, dv_ref, b_ref, o_ref):
    N, C = o_ref.shape
    NPAD = dv_ref.shape[0]
    dinv = dv_ref[...]
    rows = []
    for cc in range(C):
        sl = pl.ds(cc * NPAD, NPAD)
        t = dinv * (a0_ref[sl] + a1_ref[sl] + g_ref[sl])
        rows.append(t[None, :])
    t2 = jnp.concatenate(rows, axis=0)
    o_ref[...] = jnp.transpose(t2, (1, 0))[:N, :] + b_ref[...]

def kernel(x, edge_index, edge_weight, W, b):
    N, D = x.shape
    C = W.shape[1]
    E = edge_index.shape[1]

    GRAN = NWRIT * L * 8
    NP = -(-N // GRAN) * GRAN
    NP2 = -(-(2 * N) // GRAN) * GRAN

    eflat = edge_index.reshape(2 * E)

    dp0, dp1 = _make_deg_kernel(E, NP)(eflat, edge_weight)

    g2, dinv = pl.pallas_call(
        _tc_norm_body,
        out_shape=(
            jax.ShapeDtypeStruct((NP2,), jnp.float32),
            jax.ShapeDtypeStruct((NP,), jnp.float32),
        ),
    )(dp0, dp1, x, W)

    ap0, ap1 = _make_agg_kernel(E, NP2)(eflat, edge_weight, g2)

    out = pl.pallas_call(
        _tc_out_body,
        out_shape=jax.ShapeDtypeStruct((N, C), jnp.float32),
    )(ap0, ap1, g2, dinv, b[None, :])
    return out


# speedup vs baseline: 1.0481x; 1.0481x over previous
import functools

import jax
import jax.numpy as jnp
from jax import lax
from jax.experimental import pallas as pl
from jax.experimental.pallas import tpu as pltpu
from jax.experimental.pallas import tpu_sc as plsc

NC = 2
NS = 16
L = 16
NWRIT = 16


def _make_deg_kernel(E, NP):
    EW = E // (NC * NS)
    RED = NP // NS
    mesh = plsc.VectorSubcoreMesh(
        core_axis_name="c", subcore_axis_name="s", num_cores=NC, num_subcores=NS
    )

    @functools.partial(
        pl.kernel,
        out_type=(
            jax.ShapeDtypeStruct((NP,), jnp.float32),
            jax.ShapeDtypeStruct((NP,), jnp.float32),
        ),
        mesh=mesh,
        compiler_params=pltpu.CompilerParams(needs_layout_passes=False),
        scratch_types=[
            pltpu.VMEM((EW,), jnp.int32),
            pltpu.VMEM((EW,), jnp.float32),
            pltpu.VMEM((NP,), jnp.float32),
            pltpu.VMEM((RED,), jnp.float32),
            pltpu.VMEM((RED,), jnp.float32),
            pltpu.VMEM((RED,), jnp.float32),
            pltpu.VMEM_SHARED((NS, NP), jnp.float32),
            pltpu.SemaphoreType.DMA,
            pltpu.SemaphoreType.DMA,
        ],
    )
    def deg_kernel(ei_hbm, w_hbm, out0_hbm, out1_hbm, row_v, w_v, deg_v,
                   red_v, tmp0_v, tmp1_v, shared, sem_a, sem_b):
        c = lax.axis_index("c")
        s = lax.axis_index("s")
        wid = s * NC + c

        cp_r = pltpu.async_copy(ei_hbm.at[pl.ds(wid * EW, EW)], row_v, sem_a)
        cp_w = pltpu.async_copy(w_hbm.at[pl.ds(wid * EW, EW)], w_v, sem_b)

        @plsc.parallel_loop(0, NP // L, 1, unroll=8)
        def _(i):
            deg_v[pl.ds(i * L, L)] = jnp.zeros((L,), jnp.float32)
        cp_r.wait()
        cp_w.wait()

        @plsc.parallel_loop(0, EW // L, 1, unroll=5)
        def _(i):
            o = pl.ds(i * L, L)
            plsc.addupdate_scatter(deg_v, [row_v[o]], w_v[o])

        pltpu.sync_copy(deg_v, shared.at[s])
        plsc.subcore_barrier()

        slr = pl.ds(s * RED, RED)
        pltpu.sync_copy(shared.at[0, slr], red_v)

        def red_body(j, _):
            t = 1 + j * 2
            pltpu.async_copy(shared.at[t, slr], tmp0_v, sem_a)
            pltpu.async_copy(shared.at[t + 1, slr], tmp1_v, sem_b)
            pltpu.make_async_copy(shared.at[t, slr], tmp0_v, sem_a).wait()

            @plsc.parallel_loop(0, RED // L, 1, unroll=4)
            def _(k):
                plsc.addupdate(red_v.at[pl.ds(k * L, L)], tmp0_v[pl.ds(k * L, L)])
            pltpu.make_async_copy(shared.at[t + 1, slr], tmp1_v, sem_b).wait()

            @plsc.parallel_loop(0, RED // L, 1, unroll=4)
            def _(k):
                plsc.addupdate(red_v.at[pl.ds(k * L, L)], tmp1_v[pl.ds(k * L, L)])
            return ()
        lax.fori_loop(0, (NS - 1) // 2, red_body, ())
        pltpu.sync_copy(shared.at[NS - 1, slr], tmp0_v)

        @plsc.parallel_loop(0, RED // L, 1, unroll=4)
        def _(k):
            plsc.addupdate(red_v.at[pl.ds(k * L, L)], tmp0_v[pl.ds(k * L, L)])

        @pl.when(c == 0)
        def _():
            pltpu.sync_copy(red_v, out0_hbm.at[slr])

        @pl.when(c == 1)
        def _():
            pltpu.sync_copy(red_v, out1_hbm.at[slr])

    return deg_kernel


def _make_agg_kernel(E, NP2):
    EW = E // (NC * NS)
    RED = NP2 // NS
    mesh = plsc.VectorSubcoreMesh(
        core_axis_name="c", subcore_axis_name="s", num_cores=NC, num_subcores=NS
    )

    @functools.partial(
        pl.kernel,
        out_type=(
            jax.ShapeDtypeStruct((NP2,), jnp.float32),
            jax.ShapeDtypeStruct((NP2,), jnp.float32),
        ),
        mesh=mesh,
        compiler_params=pltpu.CompilerParams(needs_layout_passes=False),
        scratch_types=[
            pltpu.VMEM((EW,), jnp.int32),
            pltpu.VMEM((EW,), jnp.int32),
            pltpu.VMEM((EW,), jnp.float32),
            pltpu.VMEM((NP2,), jnp.float32),
            pltpu.VMEM((NP2,), jnp.float32),
            pltpu.VMEM((RED,), jnp.float32),
            pltpu.VMEM((RED,), jnp.float32),
            pltpu.VMEM((RED,), jnp.float32),
            pltpu.VMEM_SHARED((NS, NP2), jnp.float32),
            pltpu.SemaphoreType.DMA,
            pltpu.SemaphoreType.DMA,
            pltpu.SemaphoreType.DMA,
            pltpu.SemaphoreType.DMA,
        ],
    )
    def agg_kernel(ei_hbm, w_hbm, g2_hbm, out0_hbm, out1_hbm,
                   row_v, col_v, w_v, g2_v, acc_v, red_v, tmp0_v, tmp1_v,
                   shared, sem_a, sem_b, sem_c, sem_d):
        c = lax.axis_index("c")
        s = lax.axis_index("s")
        wid = s * NC + c

        cp_g = pltpu.async_copy(g2_hbm, g2_v, sem_c)
        cp_r = pltpu.async_copy(ei_hbm.at[pl.ds(wid * EW, EW)], row_v, sem_a)
        cp_c = pltpu.async_copy(ei_hbm.at[pl.ds(E + wid * EW, EW)], col_v, sem_b)
        cp_w = pltpu.async_copy(w_hbm.at[pl.ds(wid * EW, EW)], w_v, sem_d)

        @plsc.parallel_loop(0, NP2 // L, 1, unroll=8)
        def _(i):
            acc_v[pl.ds(i * L, L)] = jnp.zeros((L,), jnp.float32)
        cp_g.wait()
        cp_r.wait()
        cp_c.wait()
        cp_w.wait()

        npad = jnp.int32(NP2 // 2)

        @plsc.parallel_loop(0, EW // L, 1, unroll=5)
        def _(i):
            o = pl.ds(i * L, L)
            r = row_v[o]
            cc = col_v[o]
            wt = w_v[o]
            g0 = plsc.load_gather(g2_v, [r])
            g1 = plsc.load_gather(g2_v, [r + npad])
            plsc.addupdate_scatter(acc_v, [cc], g0 * wt)
            plsc.addupdate_scatter(acc_v, [cc + npad], g1 * wt)

        pltpu.sync_copy(acc_v, shared.at[s])
        plsc.subcore_barrier()

        slr = pl.ds(s * RED, RED)
        pltpu.sync_copy(shared.at[0, slr], red_v)

        def red_body(j, _):
            t = 1 + j * 2
            pltpu.async_copy(shared.at[t, slr], tmp0_v, sem_a)
            pltpu.async_copy(shared.at[t + 1, slr], tmp1_v, sem_b)
            pltpu.make_async_copy(shared.at[t, slr], tmp0_v, sem_a).wait()

            @plsc.parallel_loop(0, RED // L, 1, unroll=4)
            def _(k):
                plsc.addupdate(red_v.at[pl.ds(k * L, L)], tmp0_v[pl.ds(k * L, L)])
            pltpu.make_async_copy(shared.at[t + 1, slr], tmp1_v, sem_b).wait()

            @plsc.parallel_loop(0, RED // L, 1, unroll=4)
            def _(k):
                plsc.addupdate(red_v.at[pl.ds(k * L, L)], tmp1_v[pl.ds(k * L, L)])
            return ()
        lax.fori_loop(0, (NS - 1) // 2, red_body, ())
        pltpu.sync_copy(shared.at[NS - 1, slr], tmp0_v)

        @plsc.parallel_loop(0, RED // L, 1, unroll=4)
        def _(k):
            plsc.addupdate(red_v.at[pl.ds(k * L, L)], tmp0_v[pl.ds(k * L, L)])

        @pl.when(c == 0)
        def _():
            pltpu.sync_copy(red_v, out0_hbm.at[slr])

        @pl.when(c == 1)
        def _():
            pltpu.sync_copy(red_v, out1_hbm.at[slr])

    return agg_kernel


def _tc_mm_body(x_ref, w_ref, h_ref):
    NP2 = h_ref.shape[0]
    NPAD = NP2 // 2
    N = x_ref.shape[0]
    hT = lax.dot_general(
        w_ref[...], x_ref[...],
        dimension_numbers=(((0,), (1,)), ((), ())),
        preferred_element_type=jnp.float32,
    )
    C = hT.shape[0]
    for cc in range(C):
        h_ref[pl.ds(cc * NPAD, N)] = hT[cc]
        h_ref[pl.ds(cc * NPAD + N, NPAD - N)] = jnp.zeros(
            (NPAD - N,), jnp.float32)


def _tc_norm_body(d0_ref, d1_ref, h_ref, g_ref, dv_ref):
    NPAD = d0_ref.shape[0]
    deg = d0_ref[...] + d1_ref[...] + 1.0
    dinv = lax.rsqrt(jnp.maximum(deg, 1e-12))
    dv_ref[...] = dinv
    for cc in range(2):
        sl = pl.ds(cc * NPAD, NPAD)
        g_ref[sl] = h_ref[sl] * dinv


def _tc_out_body(a0_ref, a1_ref, g_ref, dv_ref, b_ref, o_ref):
    N, C = o_ref.shape
    NPAD = dv_ref.shape[0]
    dinv = dv_ref[...]
    rows = []
    for cc in range(C):
        sl = pl.ds(cc * NPAD, NPAD)
        t = dinv * (a0_ref[sl] + a1_ref[sl] + g_ref[sl])
        rows.append(t[None, :])
    t2 = jnp.concatenate(rows, axis=0)
    o_ref[...] = jnp.transpose(t2, (1, 0))[:N, :] + b_ref[...]

def kernel(x, edge_index, edge_weight, W, b):
    N, D = x.shape
    C = W.shape[1]
    E = edge_index.shape[1]

    GRAN = NWRIT * L * 8
    NP = -(-N // GRAN) * GRAN
    NP2 = -(-(2 * N) // GRAN) * GRAN

    eflat = edge_index.reshape(2 * E)

    dp0, dp1 = _make_deg_kernel(E, NP)(eflat, edge_weight)

    h2 = pl.pallas_call(
        _tc_mm_body,
        out_shape=jax.ShapeDtypeStruct((NP2,), jnp.float32),
    )(x, W)

    g2, dinv = pl.pallas_call(
        _tc_norm_body,
        out_shape=(
            jax.ShapeDtypeStruct((NP2,), jnp.float32),
            jax.ShapeDtypeStruct((NP,), jnp.float32),
        ),
    )(dp0, dp1, h2)

    ap0, ap1 = _make_agg_kernel(E, NP2)(eflat, edge_weight, g2)

    out = pl.pallas_call(
        _tc_out_body,
        out_shape=jax.ShapeDtypeStruct((N, C), jnp.float32),
    )(ap0, ap1, g2, dinv, b[None, :])
    return out
